# Initial kernel scaffold; baseline (speedup 1.0000x reference)
import jax
import jax.numpy as jnp
from jax.experimental import pallas as pl


def kernel(s, v, positions, edge_index, v_w, v_b, msg_w1, msg_b1, msg_w2, msg_b2, pos_w1, pos_b1, pos_w2, pos_b2, upd_w1, upd_b1, upd_w2, upd_b2, gp_left_w, gp_left_b, gp_right_w, gp_right_b, gp_out_w, gp_out_b, gp_norm_a):
    raise NotImplementedError("write your pallas kernel here")


# speedup vs baseline: 7.8988x; 7.8988x over previous
import functools

import jax
import jax.numpy as jnp
import numpy as np
from jax import lax
from jax.experimental import pallas as pl
from jax.experimental.pallas import tpu as pltpu
from jax.experimental.pallas import tpu_sc as plsc

N = 10000
E = 320000
NSF = 128
HID = 128
NV = 16
F = NV * 8

NC = 2
NSC = 16
NW = NC * NSC
EW = E // NW
ECH = 80
NCHUNK = EW // ECH

_SUB = np.array([1, 3, 3, 1])


def _build_cayley_np():
    blades = [0, 1, 2, 4, 3, 5, 6, 7]
    index = {b: i for i, b in enumerate(blades)}

    def reorder_sign(a, b):
        a = a >> 1
        s = 0
        while a:
            s += bin(a & b).count('1')
            a = a >> 1
        return -1.0 if (s % 2) else 1.0

    C = np.zeros((8, 8, 8), dtype=np.float32)
    for i, a in enumerate(blades):
        for k, b in enumerate(blades):
            C[i, index[a ^ b], k] += reorder_sign(a, b)
    return C


_CAY = _build_cayley_np()

_G_SUM = np.kron(np.eye(NV), np.ones((8, 1))).astype(np.float32)
_R_EXP = np.kron(np.eye(NV), np.ones((1, 8))).astype(np.float32)
_E_SEL = [np.kron(np.eye(NV), ((np.arange(8) == i).astype(np.float32)[:, None]
                               * np.ones((1, 8), np.float32))) for i in range(8)]
_M_MIX = [np.kron(np.eye(NV), _CAY[i].T).astype(np.float32) for i in range(8)]


def _mv_big(w):
    w8 = jnp.repeat(w, jnp.asarray(_SUB), axis=-1, total_repeat_length=8)
    wt = jnp.transpose(w8, (1, 0, 2))
    eye8 = jnp.eye(8, dtype=w.dtype)
    big = jnp.einsum('mni,ij->minj', wt, eye8)
    return big.reshape(w.shape[1] * 8, w.shape[0] * 8)


def _bias_flat(b):
    return jnp.zeros((b.shape[0], 8), b.dtype).at[:, 0].set(b).reshape(-1)


def _pre_body(s_ref, vf_ref, pos_ref, wsi_ref, wsj_ref, wbv_ref, wp_ref,
              b1_ref, ts_ref, tr_ref):
    s_blk = s_ref[...]
    posw = pos_ref[...] * wp_ref[...]
    pv = jnp.dot(vf_ref[...], wbv_ref[...], preferred_element_type=jnp.float32)
    ts_ref[:, :NSF] = jnp.dot(s_blk, wsi_ref[...],
                              preferred_element_type=jnp.float32) + posw
    ts_ref[:, NSF:] = pv
    tr_ref[:, :NSF] = jnp.dot(s_blk, wsj_ref[...],
                              preferred_element_type=jnp.float32) - posw + b1_ref[...]
    tr_ref[:, NSF:] = pv


def _edge_body(ts_ref, tr_ref, weg_ref, w2_ref, pw1_ref, wpr_ref, bias_ref,
               msg_ref, pm_ref):
    ts = ts_ref[...]
    tr = tr_ref[...]
    bias = bias_ref[...]
    vij = tr[:, NSF:] - ts[:, NSF:] + bias[0:1, :]
    hpre = ts[:, :NSF] + tr[:, :NSF] + jnp.dot(
        vij * vij, weg_ref[...], preferred_element_type=jnp.float32)
    h = jnp.maximum(hpre, 0.0)
    msg = jnp.dot(h, w2_ref[...], preferred_element_type=jnp.float32) + bias[1:2, :]
    ph = jnp.maximum(
        jnp.dot(msg, pw1_ref[...], preferred_element_type=jnp.float32) + bias[2:3, :],
        0.0)
    pse = jnp.dot(ph, wpr_ref[...], preferred_element_type=jnp.float32) + bias[3:4, :]
    msg_ref[...] = msg
    pm_ref[...] = vij * pse


def _node_body(s_ref, vf_ref, msgp_ref, pmp_ref, cntp_ref,
               u1a_ref, u1b_ref, u2_ref, wls_ref, wrs_ref, wo1_ref, wo2_ref,
               g_ref, vec_ref, blrs_ref, sout_ref, vout_ref):
    s_blk = s_ref[...]
    vecs = vec_ref[...]
    blrs = blrs_ref[...]
    cnt = cntp_ref[0, :, :1] + cntp_ref[1, :, :1]
    sq = jnp.sqrt(cnt)
    ma = (msgp_ref[0] + msgp_ref[1]) / sq
    pma = (pmp_ref[0] + pmp_ref[1]) / sq
    uh = jnp.maximum(
        jnp.dot(s_blk, u1a_ref[...], preferred_element_type=jnp.float32)
        + jnp.dot(ma, u1b_ref[...], preferred_element_type=jnp.float32)
        + vecs[0:1, :], 0.0)
    sout_ref[...] = s_blk + jnp.dot(
        uh, u2_ref[...], preferred_element_type=jnp.float32) + vecs[1:2, :]
    lh = jnp.dot(pma, wls_ref[...], preferred_element_type=jnp.float32) + blrs[0:1, :]
    rh = jnp.dot(pma, wrs_ref[...], preferred_element_type=jnp.float32) + blrs[1:2, :]
    gp = lh[:, :F] * rh[:, :F]
    for i in range(1, 8):
        gp = gp + lh[:, i * F:(i + 1) * F] * rh[:, i * F:(i + 1) * F]
    vo = (jnp.dot(gp, wo1_ref[...], preferred_element_type=jnp.float32)
          + jnp.dot(pma, wo2_ref[...], preferred_element_type=jnp.float32)
          + vecs[2:3, :])
    ss = jnp.dot(vo * vo, g_ref[...], preferred_element_type=jnp.float32)
    nrm = jnp.sqrt(ss + 1e-8)
    mean = jnp.sum(nrm, axis=1, keepdims=True) * (1.0 / NV) + 1e-6
    vout_ref[...] = vecs[3:4, :] * vo / mean + vf_ref[...]


def _sc_mesh():
    return plsc.VectorSubcoreMesh(core_axis_name="c", subcore_axis_name="s",
                                  num_cores=NC, num_subcores=NSC)


def _gather_call(ts, tr, send, rec):
    @functools.partial(
        pl.kernel,
        out_type=(jax.ShapeDtypeStruct((E, 2 * NSF), jnp.float32),
                  jax.ShapeDtypeStruct((E, 2 * NSF), jnp.float32)),
        mesh=_sc_mesh(),
        scratch_types=[
            pltpu.VMEM((ECH,), jnp.int32),
            pltpu.VMEM((ECH,), jnp.int32),
            pltpu.VMEM((ECH, 2 * NSF), jnp.float32),
            pltpu.VMEM((ECH, 2 * NSF), jnp.float32),
            pltpu.SemaphoreType.DMA,
            pltpu.SemaphoreType.DMA,
        ],
    )
    def k(ts_hbm, tr_hbm, send_hbm, rec_hbm, os_hbm, or_hbm,
          ids, idr, bs, br, sem1, sem2):
        wid = lax.axis_index("s") * NC + lax.axis_index("c")
        base = wid * EW

        @pl.loop(0, NCHUNK)
        def _(ci):
            off = base + ci * ECH
            pltpu.sync_copy(send_hbm.at[pl.ds(off, ECH)], ids)
            pltpu.sync_copy(rec_hbm.at[pl.ds(off, ECH)], idr)
            gs = pltpu.async_copy(ts_hbm.at[ids], bs, sem1)
            gr = pltpu.async_copy(tr_hbm.at[idr], br, sem2)
            gs.wait()
            gr.wait()
            pltpu.sync_copy(bs, os_hbm.at[pl.ds(off, ECH)])
            pltpu.sync_copy(br, or_hbm.at[pl.ds(off, ECH)])

    return k(ts, tr, send, rec)


_WB = 624


def _scatter_call(msg, pm, rec2, send2, zmsg, ones_c):
    @functools.partial(
        pl.kernel,
        out_type=(jax.ShapeDtypeStruct((NC * N, NSF), jnp.float32),
                  jax.ShapeDtypeStruct((NC * N, F), jnp.float32),
                  jax.ShapeDtypeStruct((NC * N, NSF), jnp.float32)),
        mesh=_sc_mesh(),
        scratch_types=[
            pltpu.VMEM_SHARED((N, NSF), jnp.float32),
            pltpu.VMEM((ECH, NSF), jnp.float32),
            pltpu.VMEM((ECH, NSF), jnp.float32),
            pltpu.VMEM((1, ECH), jnp.int32),
        ],
    )
    def k(msg_hbm, pm_hbm, rec_hbm, send_hbm, zm_hbm, on_hbm,
          omsg_hbm, opm_hbm, ocnt_hbm, acc, buf, ones_v, idr):
        c = lax.axis_index("c")
        sid = lax.axis_index("s")
        cbase = (sid * NC + c) * NCHUNK
        r0 = sid * _WB
        tail = NSC * _WB
        ntail = N - tail

        pltpu.sync_copy(on_hbm, ones_v)

        @pl.when(sid == 0)
        def _():
            pltpu.sync_copy(zm_hbm, acc)

        plsc.subcore_barrier()

        @pl.loop(0, NCHUNK)
        def _(ci):
            row = cbase + ci
            pltpu.sync_copy(rec_hbm.at[pl.ds(row, 1)], idr)
            pltpu.sync_copy(msg_hbm.at[pl.ds(row * ECH, ECH)], buf)
            pltpu.sync_copy(buf, acc.at[idr.at[0]], add=True)

        plsc.subcore_barrier()
        pltpu.sync_copy(acc.at[pl.ds(r0, _WB)], omsg_hbm.at[pl.ds(c * N + r0, _WB)])

        @pl.when(sid == NSC - 1)
        def _():
            pltpu.sync_copy(acc.at[pl.ds(tail, ntail)],
                            omsg_hbm.at[pl.ds(c * N + tail, ntail)])

        plsc.subcore_barrier()

        @pl.when(sid == 0)
        def _():
            pltpu.sync_copy(zm_hbm, acc)

        plsc.subcore_barrier()

        @pl.loop(0, NCHUNK)
        def _(ci):
            row = cbase + ci
            pltpu.sync_copy(rec_hbm.at[pl.ds(row, 1)], idr)
            pltpu.sync_copy(pm_hbm.at[pl.ds(row * ECH, ECH)], buf)
            pltpu.sync_copy(buf, acc.at[idr.at[0]], add=True)

        plsc.subcore_barrier()
        pltpu.sync_copy(acc.at[pl.ds(r0, _WB)], opm_hbm.at[pl.ds(c * N + r0, _WB)])

        @pl.when(sid == NSC - 1)
        def _():
            pltpu.sync_copy(acc.at[pl.ds(tail, ntail)],
                            opm_hbm.at[pl.ds(c * N + tail, ntail)])

        plsc.subcore_barrier()

        @pl.when(sid == 0)
        def _():
            pltpu.sync_copy(zm_hbm, acc)

        plsc.subcore_barrier()

        @pl.loop(0, NCHUNK)
        def _(ci):
            row = cbase + ci
            pltpu.sync_copy(send_hbm.at[pl.ds(row, 1)], idr)
            pltpu.sync_copy(ones_v, acc.at[idr.at[0]], add=True)

        plsc.subcore_barrier()
        pltpu.sync_copy(acc.at[pl.ds(r0, _WB)], ocnt_hbm.at[pl.ds(c * N + r0, _WB)])

        @pl.when(sid == NSC - 1)
        def _():
            pltpu.sync_copy(acc.at[pl.ds(tail, ntail)],
                            ocnt_hbm.at[pl.ds(c * N + tail, ntail)])

    return k(msg, pm, rec2, send2, zmsg, ones_c)


BN = 2000
BE = 2560


def _full(shape):
    return pl.BlockSpec(shape, lambda i: tuple(0 for _ in shape))


def _pre_call(s, vf, pos, wsi, wsj, wbv, wp, b1):
    grid = (N // BN,)
    return pl.pallas_call(
        _pre_body,
        grid=grid,
        in_specs=[
            pl.BlockSpec((BN, NSF), lambda i: (i, 0)),
            pl.BlockSpec((BN, F), lambda i: (i, 0)),
            pl.BlockSpec((BN, 1), lambda i: (i, 0)),
            _full((NSF, NSF)), _full((F, F)), _full((F, F)),
            _full((1, NSF)), _full((1, NSF)),
        ],
        out_specs=[
            pl.BlockSpec((BN, 2 * NSF), lambda i: (i, 0)),
            pl.BlockSpec((BN, 2 * NSF), lambda i: (i, 0)),
        ],
        out_shape=[
            jax.ShapeDtypeStruct((N, 2 * NSF), jnp.float32),
            jax.ShapeDtypeStruct((N, 2 * NSF), jnp.float32),
        ],
    )(s, vf, pos, wsi, wsj, wbv, wp, b1)


def _edge_call(ts_e, tr_e, weg, w2, pw1, wpr, bias4):
    grid = (E // BE,)
    return pl.pallas_call(
        _edge_body,
        grid=grid,
        in_specs=[
            pl.BlockSpec((BE, 2 * NSF), lambda i: (i, 0)),
            pl.BlockSpec((BE, 2 * NSF), lambda i: (i, 0)),
            _full((NSF, NSF)), _full((NSF, NSF)), _full((NSF, NSF)),
            _full((NSF, NSF)), _full((4, NSF)),
        ],
        out_specs=[
            pl.BlockSpec((BE, NSF), lambda i: (i, 0)),
            pl.BlockSpec((BE, F), lambda i: (i, 0)),
        ],
        out_shape=[
            jax.ShapeDtypeStruct((E, NSF), jnp.float32),
            jax.ShapeDtypeStruct((E, F), jnp.float32),
        ],
        compiler_params=pltpu.CompilerParams(
            dimension_semantics=("arbitrary",)),
    )(ts_e, tr_e, weg, w2, pw1, wpr, bias4)


def _node_call(s, vf, msgp, pmp, cntp, u1a, u1b, u2, wls, wrs, wo1, wo2,
               g, vec4, blrs):
    grid = (N // BN,)
    return pl.pallas_call(
        _node_body,
        grid=grid,
        in_specs=[
            pl.BlockSpec((BN, NSF), lambda i: (i, 0)),
            pl.BlockSpec((BN, F), lambda i: (i, 0)),
            pl.BlockSpec((NC, BN, NSF), lambda i: (0, i, 0)),
            pl.BlockSpec((NC, BN, F), lambda i: (0, i, 0)),
            pl.BlockSpec((NC, BN, NSF), lambda i: (0, i, 0)),
            _full((NSF, NSF)), _full((NSF, NSF)), _full((NSF, NSF)),
            _full((NSF, 8 * F)), _full((NSF, 8 * F)),
            _full((F, F)), _full((F, F)),
            _full((F, NV)), _full((4, NSF)), _full((2, 8 * F)),
        ],
        out_specs=[
            pl.BlockSpec((BN, NSF), lambda i: (i, 0)),
            pl.BlockSpec((BN, F), lambda i: (i, 0)),
        ],
        out_shape=[
            jax.ShapeDtypeStruct((N, NSF), jnp.float32),
            jax.ShapeDtypeStruct((N, F), jnp.float32),
        ],
    )(s, vf, msgp, pmp, cntp, u1a, u1b, u2, wls, wrs, wo1, wo2, g, vec4, blrs)


def kernel(s, v, positions, edge_index, v_w, v_b, msg_w1, msg_b1, msg_w2,
           msg_b2, pos_w1, pos_b1, pos_w2, pos_b2, upd_w1, upd_b1, upd_w2,
           upd_b2, gp_left_w, gp_left_b, gp_right_w, gp_right_b, gp_out_w,
           gp_out_b, gp_norm_a):
    f32 = jnp.float32
    send = edge_index[0]
    rec = edge_index[1]
    vf = v.reshape(N, F)
    pos = positions.reshape(N, 1)

    wsi = msg_w1[:, :NSF].T
    wsj = msg_w1[:, NSF:2 * NSF].T
    we_t = msg_w1[:, 2 * NSF:2 * NSF + NV].T
    wp = msg_w1[:, 2 * NSF + NV].reshape(1, NSF)
    weg = jnp.asarray(_G_SUM) @ we_t
    wbv = _mv_big(v_w)
    bv = _bias_flat(v_b)
    wpr = pos_w2.T @ jnp.asarray(_R_EXP)
    bpr = pos_b2 @ jnp.asarray(_R_EXP)
    bias4 = jnp.stack([bv, msg_b2, pos_b1, bpr], axis=0)

    wbl = _mv_big(gp_left_w)
    bl = _bias_flat(gp_left_b)
    wbr = _mv_big(gp_right_w)
    br = _bias_flat(gp_right_b)
    wls = jnp.concatenate([wbl @ jnp.asarray(m) for m in _E_SEL], axis=1)
    bls = jnp.concatenate([bl @ jnp.asarray(m) for m in _E_SEL], axis=0)
    wrs = jnp.concatenate([wbr @ jnp.asarray(m) for m in _M_MIX], axis=1)
    brs = jnp.concatenate([br @ jnp.asarray(m) for m in _M_MIX], axis=0)
    blrs = jnp.stack([bls, brs], axis=0)

    wbo = _mv_big(gp_out_w)
    wo1 = wbo[:F]
    wo2 = wbo[F:]
    bo = _bias_flat(gp_out_b)
    arep = jnp.repeat(gp_norm_a, 8)
    vec4 = jnp.stack([upd_b1, upd_b2, bo, arep], axis=0)
    u1a = upd_w1[:, :NSF].T
    u1b = upd_w1[:, NSF:].T
    u2 = upd_w2.T

    ts, tr = _pre_call(s, vf, pos, wsi, wsj, wbv, wp,
                       msg_b1.reshape(1, NSF))

    ts_e, tr_e = _gather_call(ts, tr, send, rec)

    msg_e, pm_e = _edge_call(ts_e, tr_e, weg, msg_w2.T, pos_w1.T, wpr, bias4)

    zmsg = jnp.zeros((N, NSF), f32)
    ones_c = jnp.ones((ECH, NSF), f32)
    rec2 = rec.reshape(E // ECH, ECH)
    send2 = send.reshape(E // ECH, ECH)
    msg_part, pm_part, cnt_part = _scatter_call(
        msg_e, pm_e, rec2, send2, zmsg, ones_c)

    s_out, v_out = _node_call(
        s, vf,
        msg_part.reshape(NC, N, NSF),
        pm_part.reshape(NC, N, F),
        cnt_part.reshape(NC, N, NSF),
        u1a, u1b, u2, wls, wrs, wo1, wo2,
        jnp.asarray(_G_SUM), vec4, blrs)

    return (s_out, v_out.reshape(N, NV, 8))


# speedup vs baseline: 13.8474x; 1.7531x over previous
import functools

import jax
import jax.numpy as jnp
import numpy as np
from jax import lax
from jax.experimental import pallas as pl
from jax.experimental.pallas import tpu as pltpu
from jax.experimental.pallas import tpu_sc as plsc

N = 10000
E = 320000
NSF = 128
HID = 128
NV = 16
F = NV * 8

NC = 2
NSC = 16
NW = NC * NSC
EW = E // NW
ECH = 80
NCHUNK = EW // ECH

_SUB = np.array([1, 3, 3, 1])


def _build_cayley_np():
    blades = [0, 1, 2, 4, 3, 5, 6, 7]
    index = {b: i for i, b in enumerate(blades)}

    def reorder_sign(a, b):
        a = a >> 1
        s = 0
        while a:
            s += bin(a & b).count('1')
            a = a >> 1
        return -1.0 if (s % 2) else 1.0

    C = np.zeros((8, 8, 8), dtype=np.float32)
    for i, a in enumerate(blades):
        for k, b in enumerate(blades):
            C[i, index[a ^ b], k] += reorder_sign(a, b)
    return C


_CAY = _build_cayley_np()

_G_SUM = np.kron(np.eye(NV), np.ones((8, 1))).astype(np.float32)
_R_EXP = np.kron(np.eye(NV), np.ones((1, 8))).astype(np.float32)
_E_SEL = [np.kron(np.eye(NV), ((np.arange(8) == i).astype(np.float32)[:, None]
                               * np.ones((1, 8), np.float32))) for i in range(8)]
_M_MIX = [np.kron(np.eye(NV), _CAY[i].T).astype(np.float32) for i in range(8)]


def _mv_big(w):
    w8 = jnp.repeat(w, jnp.asarray(_SUB), axis=-1, total_repeat_length=8)
    wt = jnp.transpose(w8, (1, 0, 2))
    eye8 = jnp.eye(8, dtype=w.dtype)
    big = jnp.einsum('mni,ij->minj', wt, eye8)
    return big.reshape(w.shape[1] * 8, w.shape[0] * 8)


def _bias_flat(b):
    return jnp.zeros((b.shape[0], 8), b.dtype).at[:, 0].set(b).reshape(-1)


def _pre_body(s_ref, vf_ref, pos_ref, wsi_ref, wsj_ref, wbv_ref, wp_ref,
              b1_ref, ts_ref, tr_ref):
    s_blk = s_ref[...]
    posw = pos_ref[...] * wp_ref[...]
    pv = jnp.dot(vf_ref[...], wbv_ref[...], preferred_element_type=jnp.float32)
    a = jnp.dot(s_blk, wsi_ref[...], preferred_element_type=jnp.float32) + posw
    b = (jnp.dot(s_blk, wsj_ref[...], preferred_element_type=jnp.float32)
         - posw + b1_ref[...])

    def _rne16(x):
        u = lax.bitcast_convert_type(x, jnp.uint32)
        return (u + 0x7FFF + ((u >> 16) & 1)) >> 16

    pv_hi = _rne16(pv) << 16
    ts_ref[...] = lax.bitcast_convert_type(pv_hi | _rne16(a), jnp.int32)
    tr_ref[...] = lax.bitcast_convert_type(pv_hi | _rne16(b), jnp.int32)


def _edge_body(ts_ref, tr_ref, weg_ref, w2_ref, pw1_ref, wpr_ref, bias_ref,
               msg_ref, pm_ref):
    tsu = lax.bitcast_convert_type(ts_ref[...], jnp.uint32)
    tru = lax.bitcast_convert_type(tr_ref[...], jnp.uint32)
    a_s = lax.bitcast_convert_type(tsu << 16, jnp.float32)
    b_r = lax.bitcast_convert_type(tru << 16, jnp.float32)
    pv_s = lax.bitcast_convert_type(tsu & jnp.uint32(0xFFFF0000), jnp.float32)
    pv_r = lax.bitcast_convert_type(tru & jnp.uint32(0xFFFF0000), jnp.float32)
    bias = bias_ref[...]
    vij = pv_r - pv_s + bias[0:1, :]
    vsq = (vij * vij).astype(jnp.bfloat16)
    hpre = (a_s + b_r
            + jnp.dot(vsq, weg_ref[...], preferred_element_type=jnp.float32))
    h = jnp.maximum(hpre, 0.0).astype(jnp.bfloat16)
    msg = jnp.dot(h, w2_ref[...], preferred_element_type=jnp.float32) + bias[1:2, :]
    ph = jnp.maximum(
        jnp.dot(msg.astype(jnp.bfloat16), pw1_ref[...],
                preferred_element_type=jnp.float32) + bias[2:3, :], 0.0)
    pse = jnp.dot(ph.astype(jnp.bfloat16), wpr_ref[...],
                  preferred_element_type=jnp.float32) + bias[3:4, :]
    msg_ref[...] = msg
    pm_ref[...] = vij * pse


def _node_body(s_ref, vf_ref, msgp_ref, pmp_ref, cntp_ref,
               u1a_ref, u1b_ref, u2_ref, wls_ref, wrs_ref, wo1_ref, wo2_ref,
               g_ref, vec_ref, blrs_ref, sout_ref, vout_ref):
    s_blk = s_ref[...]
    vecs = vec_ref[...]
    blrs = blrs_ref[...]
    cnt = cntp_ref[0, :, :1] + cntp_ref[1, :, :1]
    sq = jnp.sqrt(cnt)
    ma = (msgp_ref[0] + msgp_ref[1]) / sq
    pma = (pmp_ref[0] + pmp_ref[1]) / sq
    uh = jnp.maximum(
        jnp.dot(s_blk, u1a_ref[...], preferred_element_type=jnp.float32)
        + jnp.dot(ma, u1b_ref[...], preferred_element_type=jnp.float32)
        + vecs[0:1, :], 0.0)
    sout_ref[...] = s_blk + jnp.dot(
        uh, u2_ref[...], preferred_element_type=jnp.float32) + vecs[1:2, :]
    lh = jnp.dot(pma, wls_ref[...], preferred_element_type=jnp.float32) + blrs[0:1, :]
    rh = jnp.dot(pma, wrs_ref[...], preferred_element_type=jnp.float32) + blrs[1:2, :]
    gp = lh[:, :F] * rh[:, :F]
    for i in range(1, 8):
        gp = gp + lh[:, i * F:(i + 1) * F] * rh[:, i * F:(i + 1) * F]
    vo = (jnp.dot(gp, wo1_ref[...], preferred_element_type=jnp.float32)
          + jnp.dot(pma, wo2_ref[...], preferred_element_type=jnp.float32)
          + vecs[2:3, :])
    ss = jnp.dot(vo * vo, g_ref[...], preferred_element_type=jnp.float32)
    nrm = jnp.sqrt(ss + 1e-8)
    mean = jnp.sum(nrm, axis=1, keepdims=True) * (1.0 / NV) + 1e-6
    vout_ref[...] = vecs[3:4, :] * vo / mean + vf_ref[...]


def _sc_mesh():
    return plsc.VectorSubcoreMesh(core_axis_name="c", subcore_axis_name="s",
                                  num_cores=NC, num_subcores=NSC)


def _gather_call(ts, tr, send, rec):
    @functools.partial(
        pl.kernel,
        out_type=(jax.ShapeDtypeStruct((E, NSF), jnp.int32),
                  jax.ShapeDtypeStruct((E, NSF), jnp.int32)),
        mesh=_sc_mesh(),
        scratch_types=[
            pltpu.VMEM((EW,), jnp.int32),
            pltpu.VMEM((EW,), jnp.int32),
            pltpu.VMEM((ECH, NSF), jnp.int32),
            pltpu.VMEM((ECH, NSF), jnp.int32),
            pltpu.VMEM((ECH, NSF), jnp.int32),
            pltpu.VMEM((ECH, NSF), jnp.int32),
            pltpu.SemaphoreType.DMA,
            pltpu.SemaphoreType.DMA,
        ],
    )
    def k(ts_hbm, tr_hbm, send_hbm, rec_hbm, os_hbm, or_hbm,
          slab_s, slab_r, bs0, br0, bs1, br1, sg, sw):
        wid = lax.axis_index("s") * NC + lax.axis_index("c")
        base = wid * EW
        pltpu.sync_copy(send_hbm.at[pl.ds(base, EW)], slab_s)
        pltpu.sync_copy(rec_hbm.at[pl.ds(base, EW)], slab_r)

        @pl.loop(0, NCHUNK // 2)
        def _(i):
            e0 = 2 * i * ECH
            e1 = e0 + ECH
            g0a = pltpu.async_copy(ts_hbm.at[slab_s.at[pl.ds(e0, ECH)]], bs0, sg)
            g0b = pltpu.async_copy(tr_hbm.at[slab_r.at[pl.ds(e0, ECH)]], br0, sg)
            g1a = pltpu.async_copy(ts_hbm.at[slab_s.at[pl.ds(e1, ECH)]], bs1, sg)
            g1b = pltpu.async_copy(tr_hbm.at[slab_r.at[pl.ds(e1, ECH)]], br1, sg)
            g0a.wait()
            g0b.wait()
            w0a = pltpu.async_copy(bs0, os_hbm.at[pl.ds(base + e0, ECH)], sw)
            w0b = pltpu.async_copy(br0, or_hbm.at[pl.ds(base + e0, ECH)], sw)
            g1a.wait()
            g1b.wait()
            w1a = pltpu.async_copy(bs1, os_hbm.at[pl.ds(base + e1, ECH)], sw)
            w1b = pltpu.async_copy(br1, or_hbm.at[pl.ds(base + e1, ECH)], sw)
            w0a.wait()
            w0b.wait()
            w1a.wait()
            w1b.wait()

        elast = (NCHUNK - 1) * ECH
        ga = pltpu.async_copy(ts_hbm.at[slab_s.at[pl.ds(elast, ECH)]], bs0, sg)
        gb = pltpu.async_copy(tr_hbm.at[slab_r.at[pl.ds(elast, ECH)]], br0, sg)
        ga.wait()
        gb.wait()
        pltpu.sync_copy(bs0, os_hbm.at[pl.ds(base + elast, ECH)])
        pltpu.sync_copy(br0, or_hbm.at[pl.ds(base + elast, ECH)])

    return k(ts, tr, send, rec)


_WB = 624


def _scatter_call(msg, pm, rec2, send2, zmsg, ones_c):
    @functools.partial(
        pl.kernel,
        out_type=(jax.ShapeDtypeStruct((NC * N, NSF), jnp.float32),
                  jax.ShapeDtypeStruct((NC * N, F), jnp.float32),
                  jax.ShapeDtypeStruct((NC * N, NSF), jnp.float32)),
        mesh=_sc_mesh(),
        scratch_types=[
            pltpu.VMEM_SHARED((N, NSF), jnp.float32),
            pltpu.VMEM((ECH, NSF), jnp.float32),
            pltpu.VMEM((ECH, NSF), jnp.float32),
            pltpu.VMEM((ECH, NSF), jnp.float32),
            pltpu.VMEM((1, ECH), jnp.int32),
            pltpu.VMEM((1, ECH), jnp.int32),
            pltpu.SemaphoreType.DMA,
            pltpu.SemaphoreType.DMA,
            pltpu.SemaphoreType.DMA,
        ],
    )
    def k(msg_hbm, pm_hbm, rec_hbm, send_hbm, zm_hbm, on_hbm,
          omsg_hbm, opm_hbm, ocnt_hbm, acc, buf0, buf1, ones_v,
          idx0, idx1, si, sl, sa):
        c = lax.axis_index("c")
        sid = lax.axis_index("s")
        cbase = (sid * NC + c) * NCHUNK
        r0 = sid * _WB
        tail = NSC * _WB
        ntail = N - tail

        pltpu.sync_copy(on_hbm, ones_v)

        def scatter_phase(src_hbm, out_hbm):
            @pl.when(sid == 0)
            def _():
                pltpu.sync_copy(zm_hbm, acc)

            plsc.subcore_barrier()

            @pl.loop(0, NCHUNK // 2)
            def _(i):
                c0 = cbase + 2 * i
                c1 = c0 + 1
                i0 = pltpu.async_copy(rec_hbm.at[pl.ds(c0, 1)], idx0, si)
                i1 = pltpu.async_copy(rec_hbm.at[pl.ds(c1, 1)], idx1, si)
                l0 = pltpu.async_copy(src_hbm.at[pl.ds(c0 * ECH, ECH)], buf0, sl)
                l1 = pltpu.async_copy(src_hbm.at[pl.ds(c1 * ECH, ECH)], buf1, sl)
                i0.wait()
                l0.wait()
                a0 = pltpu.async_copy(buf0, acc.at[idx0.at[0]], sa, add=True)
                i1.wait()
                l1.wait()
                a1 = pltpu.async_copy(buf1, acc.at[idx1.at[0]], sa, add=True)
                a0.wait()
                a1.wait()

            clast = cbase + NCHUNK - 1
            pltpu.sync_copy(rec_hbm.at[pl.ds(clast, 1)], idx0)
            pltpu.sync_copy(src_hbm.at[pl.ds(clast * ECH, ECH)], buf0)
            pltpu.sync_copy(buf0, acc.at[idx0.at[0]], add=True)

            plsc.subcore_barrier()
            pltpu.sync_copy(acc.at[pl.ds(r0, _WB)],
                            out_hbm.at[pl.ds(c * N + r0, _WB)])

            @pl.when(sid == NSC - 1)
            def _():
                pltpu.sync_copy(acc.at[pl.ds(tail, ntail)],
                                out_hbm.at[pl.ds(c * N + tail, ntail)])

            plsc.subcore_barrier()

        scatter_phase(msg_hbm, omsg_hbm)
        scatter_phase(pm_hbm, opm_hbm)

        @pl.when(sid == 0)
        def _():
            pltpu.sync_copy(zm_hbm, acc)

        plsc.subcore_barrier()

        @pl.loop(0, NCHUNK // 2)
        def _(i):
            c0 = cbase + 2 * i
            c1 = c0 + 1
            i0 = pltpu.async_copy(send_hbm.at[pl.ds(c0, 1)], idx0, si)
            i1 = pltpu.async_copy(send_hbm.at[pl.ds(c1, 1)], idx1, si)
            i0.wait()
            a0 = pltpu.async_copy(ones_v, acc.at[idx0.at[0]], sa, add=True)
            i1.wait()
            a1 = pltpu.async_copy(ones_v, acc.at[idx1.at[0]], sa, add=True)
            a0.wait()
            a1.wait()

        clast = cbase + NCHUNK - 1
        pltpu.sync_copy(send_hbm.at[pl.ds(clast, 1)], idx0)
        pltpu.sync_copy(ones_v, acc.at[idx0.at[0]], add=True)

        plsc.subcore_barrier()
        pltpu.sync_copy(acc.at[pl.ds(r0, _WB)], ocnt_hbm.at[pl.ds(c * N + r0, _WB)])

        @pl.when(sid == NSC - 1)
        def _():
            pltpu.sync_copy(acc.at[pl.ds(tail, ntail)],
                            ocnt_hbm.at[pl.ds(c * N + tail, ntail)])

    return k(msg, pm, rec2, send2, zmsg, ones_c)


BN = 2000
BE = 2560


def _full(shape):
    return pl.BlockSpec(shape, lambda i: tuple(0 for _ in shape))


def _pre_call(s, vf, pos, wsi, wsj, wbv, wp, b1):
    grid = (N // BN,)
    return pl.pallas_call(
        _pre_body,
        grid=grid,
        in_specs=[
            pl.BlockSpec((BN, NSF), lambda i: (i, 0)),
            pl.BlockSpec((BN, F), lambda i: (i, 0)),
            pl.BlockSpec((BN, 1), lambda i: (i, 0)),
            _full((NSF, NSF)), _full((F, F)), _full((F, F)),
            _full((1, NSF)), _full((1, NSF)),
        ],
        out_specs=[
            pl.BlockSpec((BN, NSF), lambda i: (i, 0)),
            pl.BlockSpec((BN, NSF), lambda i: (i, 0)),
        ],
        out_shape=[
            jax.ShapeDtypeStruct((N, NSF), jnp.int32),
            jax.ShapeDtypeStruct((N, NSF), jnp.int32),
        ],
    )(s, vf, pos, wsi, wsj, wbv, wp, b1)


def _edge_call(ts_e, tr_e, weg, w2, pw1, wpr, bias4):
    grid = (E // BE,)
    return pl.pallas_call(
        _edge_body,
        grid=grid,
        in_specs=[
            pl.BlockSpec((BE, NSF), lambda i: (i, 0)),
            pl.BlockSpec((BE, NSF), lambda i: (i, 0)),
            _full((NSF, NSF)), _full((NSF, NSF)), _full((NSF, NSF)),
            _full((NSF, NSF)), _full((4, NSF)),
        ],
        out_specs=[
            pl.BlockSpec((BE, NSF), lambda i: (i, 0)),
            pl.BlockSpec((BE, F), lambda i: (i, 0)),
        ],
        out_shape=[
            jax.ShapeDtypeStruct((E, NSF), jnp.float32),
            jax.ShapeDtypeStruct((E, F), jnp.float32),
        ],
        compiler_params=pltpu.CompilerParams(
            dimension_semantics=("arbitrary",)),
    )(ts_e, tr_e, weg, w2, pw1, wpr, bias4)


def _node_call(s, vf, msgp, pmp, cntp, u1a, u1b, u2, wls, wrs, wo1, wo2,
               g, vec4, blrs):
    grid = (N // BN,)
    return pl.pallas_call(
        _node_body,
        grid=grid,
        in_specs=[
            pl.BlockSpec((BN, NSF), lambda i: (i, 0)),
            pl.BlockSpec((BN, F), lambda i: (i, 0)),
            pl.BlockSpec((NC, BN, NSF), lambda i: (0, i, 0)),
            pl.BlockSpec((NC, BN, F), lambda i: (0, i, 0)),
            pl.BlockSpec((NC, BN, NSF), lambda i: (0, i, 0)),
            _full((NSF, NSF)), _full((NSF, NSF)), _full((NSF, NSF)),
            _full((NSF, 8 * F)), _full((NSF, 8 * F)),
            _full((F, F)), _full((F, F)),
            _full((F, NV)), _full((4, NSF)), _full((2, 8 * F)),
        ],
        out_specs=[
            pl.BlockSpec((BN, NSF), lambda i: (i, 0)),
            pl.BlockSpec((BN, F), lambda i: (i, 0)),
        ],
        out_shape=[
            jax.ShapeDtypeStruct((N, NSF), jnp.float32),
            jax.ShapeDtypeStruct((N, F), jnp.float32),
        ],
    )(s, vf, msgp, pmp, cntp, u1a, u1b, u2, wls, wrs, wo1, wo2, g, vec4, blrs)


def kernel(s, v, positions, edge_index, v_w, v_b, msg_w1, msg_b1, msg_w2,
           msg_b2, pos_w1, pos_b1, pos_w2, pos_b2, upd_w1, upd_b1, upd_w2,
           upd_b2, gp_left_w, gp_left_b, gp_right_w, gp_right_b, gp_out_w,
           gp_out_b, gp_norm_a):
    f32 = jnp.float32
    send = edge_index[0]
    rec = edge_index[1]
    vf = v.reshape(N, F)
    pos = positions.reshape(N, 1)

    wsi = msg_w1[:, :NSF].T
    wsj = msg_w1[:, NSF:2 * NSF].T
    we_t = msg_w1[:, 2 * NSF:2 * NSF + NV].T
    wp = msg_w1[:, 2 * NSF + NV].reshape(1, NSF)
    weg = jnp.asarray(_G_SUM) @ we_t
    wbv = _mv_big(v_w)
    bv = _bias_flat(v_b)
    wpr = pos_w2.T @ jnp.asarray(_R_EXP)
    bpr = pos_b2 @ jnp.asarray(_R_EXP)
    bias4 = jnp.stack([bv, msg_b2, pos_b1, bpr], axis=0)

    wbl = _mv_big(gp_left_w)
    bl = _bias_flat(gp_left_b)
    wbr = _mv_big(gp_right_w)
    br = _bias_flat(gp_right_b)
    wls = jnp.concatenate([wbl @ jnp.asarray(m) for m in _E_SEL], axis=1)
    bls = jnp.concatenate([bl @ jnp.asarray(m) for m in _E_SEL], axis=0)
    wrs = jnp.concatenate([wbr @ jnp.asarray(m) for m in _M_MIX], axis=1)
    brs = jnp.concatenate([br @ jnp.asarray(m) for m in _M_MIX], axis=0)
    blrs = jnp.stack([bls, brs], axis=0)

    wbo = _mv_big(gp_out_w)
    wo1 = wbo[:F]
    wo2 = wbo[F:]
    bo = _bias_flat(gp_out_b)
    arep = jnp.repeat(gp_norm_a, 8)
    vec4 = jnp.stack([upd_b1, upd_b2, bo, arep], axis=0)
    u1a = upd_w1[:, :NSF].T
    u1b = upd_w1[:, NSF:].T
    u2 = upd_w2.T

    ts, tr = _pre_call(s, vf, pos, wsi, wsj, wbv, wp,
                       msg_b1.reshape(1, NSF))

    rec2 = rec.reshape(E // ECH, ECH)
    send2 = send.reshape(E // ECH, ECH)
    ts_e, tr_e = _gather_call(ts, tr, send, rec)

    bf16 = jnp.bfloat16
    msg_e, pm_e = _edge_call(ts_e, tr_e, weg.astype(bf16),
                             msg_w2.T.astype(bf16), pos_w1.T.astype(bf16),
                             wpr.astype(bf16), bias4)

    zmsg = jnp.zeros((N, NSF), f32)
    ones_c = jnp.ones((ECH, NSF), f32)
    msg_part, pm_part, cnt_part = _scatter_call(
        msg_e, pm_e, rec2, send2, zmsg, ones_c)

    s_out, v_out = _node_call(
        s, vf,
        msg_part.reshape(NC, N, NSF),
        pm_part.reshape(NC, N, F),
        cnt_part.reshape(NC, N, NSF),
        u1a, u1b, u2, wls, wrs, wo1, wo2,
        jnp.asarray(_G_SUM), vec4, blrs)

    return (s_out, v_out.reshape(N, NV, 8))


# speedup vs baseline: 13.8675x; 1.0015x over previous
import functools

import jax
import jax.numpy as jnp
import numpy as np
from jax import lax
from jax.experimental import pallas as pl
from jax.experimental.pallas import tpu as pltpu
from jax.experimental.pallas import tpu_sc as plsc

N = 10000
E = 320000
NSF = 128
HID = 128
NV = 16
F = NV * 8

NC = 2
NSC = 16
NW = NC * NSC
EW = E // NW
ECH = 80
NCHUNK = EW // ECH

_SUB = np.array([1, 3, 3, 1])


def _build_cayley_np():
    blades = [0, 1, 2, 4, 3, 5, 6, 7]
    index = {b: i for i, b in enumerate(blades)}

    def reorder_sign(a, b):
        a = a >> 1
        s = 0
        while a:
            s += bin(a & b).count('1')
            a = a >> 1
        return -1.0 if (s % 2) else 1.0

    C = np.zeros((8, 8, 8), dtype=np.float32)
    for i, a in enumerate(blades):
        for k, b in enumerate(blades):
            C[i, index[a ^ b], k] += reorder_sign(a, b)
    return C


_CAY = _build_cayley_np()

_G_SUM = np.kron(np.eye(NV), np.ones((8, 1))).astype(np.float32)
_R_EXP = np.kron(np.eye(NV), np.ones((1, 8))).astype(np.float32)
_E_SEL = np.stack(
    [np.kron(np.eye(NV), ((np.arange(8) == i).astype(np.float32)[:, None]
                          * np.ones((1, 8), np.float32))) for i in range(8)])
_M_MIX = np.stack(
    [np.kron(np.eye(NV), _CAY[i].T).astype(np.float32) for i in range(8)])


def _mv_big(w):
    w8 = jnp.repeat(w, jnp.asarray(_SUB), axis=-1, total_repeat_length=8)
    wt = jnp.transpose(w8, (1, 0, 2))
    eye8 = jnp.eye(8, dtype=w.dtype)
    big = jnp.einsum('mni,ij->minj', wt, eye8)
    return big.reshape(w.shape[1] * 8, w.shape[0] * 8)


def _bias_flat(b):
    return jnp.zeros((b.shape[0], 8), b.dtype).at[:, 0].set(b).reshape(-1)


def _pre_body(s_ref, vf_ref, pos_ref, wsi_ref, wsj_ref, wbv_ref, wp_ref,
              b1_ref, ts_ref, tr_ref):
    s_blk = s_ref[...]
    posw = pos_ref[...] * wp_ref[...]
    pv = jnp.dot(vf_ref[...], wbv_ref[...], preferred_element_type=jnp.float32)
    a = jnp.dot(s_blk, wsi_ref[...], preferred_element_type=jnp.float32) + posw
    b = (jnp.dot(s_blk, wsj_ref[...], preferred_element_type=jnp.float32)
         - posw + b1_ref[...])

    def _rne16(x):
        u = lax.bitcast_convert_type(x, jnp.uint32)
        return (u + 0x7FFF + ((u >> 16) & 1)) >> 16

    pv_hi = _rne16(pv) << 16
    ts_ref[...] = lax.bitcast_convert_type(pv_hi | _rne16(a), jnp.int32)
    tr_ref[...] = lax.bitcast_convert_type(pv_hi | _rne16(b), jnp.int32)


def _edge_body(ts_ref, tr_ref, weg_ref, w2_ref, pw1_ref, wpr_ref, bias_ref,
               msg_ref, pm_ref):
    tsu = lax.bitcast_convert_type(ts_ref[...], jnp.uint32)
    tru = lax.bitcast_convert_type(tr_ref[...], jnp.uint32)
    a_s = lax.bitcast_convert_type(tsu << 16, jnp.float32)
    b_r = lax.bitcast_convert_type(tru << 16, jnp.float32)
    pv_s = lax.bitcast_convert_type(tsu & jnp.uint32(0xFFFF0000), jnp.float32)
    pv_r = lax.bitcast_convert_type(tru & jnp.uint32(0xFFFF0000), jnp.float32)
    bias = bias_ref[...]
    vij = pv_r - pv_s + bias[0:1, :]
    vsq = (vij * vij).astype(jnp.bfloat16)
    hpre = (a_s + b_r
            + jnp.dot(vsq, weg_ref[...], preferred_element_type=jnp.float32))
    h = jnp.maximum(hpre, 0.0).astype(jnp.bfloat16)
    msg = jnp.dot(h, w2_ref[...], preferred_element_type=jnp.float32) + bias[1:2, :]
    ph = jnp.maximum(
        jnp.dot(msg.astype(jnp.bfloat16), pw1_ref[...],
                preferred_element_type=jnp.float32) + bias[2:3, :], 0.0)
    pse = jnp.dot(ph.astype(jnp.bfloat16), wpr_ref[...],
                  preferred_element_type=jnp.float32) + bias[3:4, :]
    msg_ref[...] = msg
    pm_ref[...] = vij * pse


def _node_body(s_ref, vf_ref, msgp_ref, pmp_ref, cntp_ref,
               u1a_ref, u1b_ref, u2_ref, wls_ref, wrs_ref, wo1_ref, wo2_ref,
               g_ref, vec_ref, blrs_ref, sout_ref, vout_ref):
    s_blk = s_ref[...]
    vecs = vec_ref[...]
    blrs = blrs_ref[...]
    cnt = cntp_ref[0, :, :1] + cntp_ref[1, :, :1]
    sq = jnp.sqrt(cnt)
    ma = (msgp_ref[0] + msgp_ref[1]) / sq
    pma = (pmp_ref[0] + pmp_ref[1]) / sq
    uh = jnp.maximum(
        jnp.dot(s_blk, u1a_ref[...], preferred_element_type=jnp.float32)
        + jnp.dot(ma, u1b_ref[...], preferred_element_type=jnp.float32)
        + vecs[0:1, :], 0.0)
    sout_ref[...] = s_blk + jnp.dot(
        uh, u2_ref[...], preferred_element_type=jnp.float32) + vecs[1:2, :]
    lh = jnp.dot(pma, wls_ref[...], preferred_element_type=jnp.float32) + blrs[0:1, :]
    rh = jnp.dot(pma, wrs_ref[...], preferred_element_type=jnp.float32) + blrs[1:2, :]
    gp = lh[:, :F] * rh[:, :F]
    for i in range(1, 8):
        gp = gp + lh[:, i * F:(i + 1) * F] * rh[:, i * F:(i + 1) * F]
    vo = (jnp.dot(gp, wo1_ref[...], preferred_element_type=jnp.float32)
          + jnp.dot(pma, wo2_ref[...], preferred_element_type=jnp.float32)
          + vecs[2:3, :])
    ss = jnp.dot(vo * vo, g_ref[...], preferred_element_type=jnp.float32)
    nrm = jnp.sqrt(ss + 1e-8)
    mean = jnp.sum(nrm, axis=1, keepdims=True) * (1.0 / NV) + 1e-6
    vout_ref[...] = vecs[3:4, :] * vo / mean + vf_ref[...]


def _sc_mesh():
    return plsc.VectorSubcoreMesh(core_axis_name="c", subcore_axis_name="s",
                                  num_cores=NC, num_subcores=NSC)


def _gather_call(ts, tr, send, rec):
    @functools.partial(
        pl.kernel,
        out_type=(jax.ShapeDtypeStruct((E, NSF), jnp.int32),
                  jax.ShapeDtypeStruct((E, NSF), jnp.int32)),
        mesh=_sc_mesh(),
        scratch_types=[
            pltpu.VMEM((EW,), jnp.int32),
            pltpu.VMEM((EW,), jnp.int32),
            pltpu.VMEM((ECH, NSF), jnp.int32),
            pltpu.VMEM((ECH, NSF), jnp.int32),
            pltpu.VMEM((ECH, NSF), jnp.int32),
            pltpu.VMEM((ECH, NSF), jnp.int32),
            pltpu.SemaphoreType.DMA,
            pltpu.SemaphoreType.DMA,
        ],
    )
    def k(ts_hbm, tr_hbm, send_hbm, rec_hbm, os_hbm, or_hbm,
          slab_s, slab_r, bs0, br0, bs1, br1, sg, sw):
        wid = lax.axis_index("s") * NC + lax.axis_index("c")
        base = wid * EW
        pltpu.sync_copy(send_hbm.at[pl.ds(base, EW)], slab_s)
        pltpu.sync_copy(rec_hbm.at[pl.ds(base, EW)], slab_r)

        @pl.loop(0, NCHUNK // 2)
        def _(i):
            e0 = 2 * i * ECH
            e1 = e0 + ECH
            g0a = pltpu.async_copy(ts_hbm.at[slab_s.at[pl.ds(e0, ECH)]], bs0, sg)
            g0b = pltpu.async_copy(tr_hbm.at[slab_r.at[pl.ds(e0, ECH)]], br0, sg)
            g1a = pltpu.async_copy(ts_hbm.at[slab_s.at[pl.ds(e1, ECH)]], bs1, sg)
            g1b = pltpu.async_copy(tr_hbm.at[slab_r.at[pl.ds(e1, ECH)]], br1, sg)
            g0a.wait()
            g0b.wait()
            w0a = pltpu.async_copy(bs0, os_hbm.at[pl.ds(base + e0, ECH)], sw)
            w0b = pltpu.async_copy(br0, or_hbm.at[pl.ds(base + e0, ECH)], sw)
            g1a.wait()
            g1b.wait()
            w1a = pltpu.async_copy(bs1, os_hbm.at[pl.ds(base + e1, ECH)], sw)
            w1b = pltpu.async_copy(br1, or_hbm.at[pl.ds(base + e1, ECH)], sw)
            w0a.wait()
            w0b.wait()
            w1a.wait()
            w1b.wait()

        elast = (NCHUNK - 1) * ECH
        ga = pltpu.async_copy(ts_hbm.at[slab_s.at[pl.ds(elast, ECH)]], bs0, sg)
        gb = pltpu.async_copy(tr_hbm.at[slab_r.at[pl.ds(elast, ECH)]], br0, sg)
        ga.wait()
        gb.wait()
        pltpu.sync_copy(bs0, os_hbm.at[pl.ds(base + elast, ECH)])
        pltpu.sync_copy(br0, or_hbm.at[pl.ds(base + elast, ECH)])

    return k(ts, tr, send, rec)


_WB = 624


def _scatter_call(msg, pm, rec2, send2, zmsg, ones_c):
    @functools.partial(
        pl.kernel,
        out_type=(jax.ShapeDtypeStruct((NC * N, NSF), jnp.float32),
                  jax.ShapeDtypeStruct((NC * N, F), jnp.float32),
                  jax.ShapeDtypeStruct((NC * N, NSF), jnp.float32)),
        mesh=_sc_mesh(),
        scratch_types=[
            pltpu.VMEM_SHARED((N, NSF), jnp.float32),
            pltpu.VMEM((ECH, NSF), jnp.float32),
            pltpu.VMEM((ECH, NSF), jnp.float32),
            pltpu.VMEM((ECH, NSF), jnp.float32),
            pltpu.VMEM((1, ECH), jnp.int32),
            pltpu.VMEM((1, ECH), jnp.int32),
            pltpu.SemaphoreType.DMA,
            pltpu.SemaphoreType.DMA,
            pltpu.SemaphoreType.DMA,
        ],
    )
    def k(msg_hbm, pm_hbm, rec_hbm, send_hbm, zm_hbm, on_hbm,
          omsg_hbm, opm_hbm, ocnt_hbm, acc, buf0, buf1, ones_v,
          idx0, idx1, si, sl, sa):
        c = lax.axis_index("c")
        sid = lax.axis_index("s")
        cbase = (sid * NC + c) * NCHUNK
        r0 = sid * _WB
        tail = NSC * _WB
        ntail = N - tail

        pltpu.sync_copy(on_hbm, ones_v)

        def scatter_phase(src_hbm, out_hbm):
            @pl.when(sid == 0)
            def _():
                pltpu.sync_copy(zm_hbm, acc)

            plsc.subcore_barrier()

            @pl.loop(0, NCHUNK // 2)
            def _(i):
                c0 = cbase + 2 * i
                c1 = c0 + 1
                i0 = pltpu.async_copy(rec_hbm.at[pl.ds(c0, 1)], idx0, si)
                i1 = pltpu.async_copy(rec_hbm.at[pl.ds(c1, 1)], idx1, si)
                l0 = pltpu.async_copy(src_hbm.at[pl.ds(c0 * ECH, ECH)], buf0, sl)
                l1 = pltpu.async_copy(src_hbm.at[pl.ds(c1 * ECH, ECH)], buf1, sl)
                i0.wait()
                l0.wait()
                a0 = pltpu.async_copy(buf0, acc.at[idx0.at[0]], sa, add=True)
                i1.wait()
                l1.wait()
                a1 = pltpu.async_copy(buf1, acc.at[idx1.at[0]], sa, add=True)
                a0.wait()
                a1.wait()

            clast = cbase + NCHUNK - 1
            pltpu.sync_copy(rec_hbm.at[pl.ds(clast, 1)], idx0)
            pltpu.sync_copy(src_hbm.at[pl.ds(clast * ECH, ECH)], buf0)
            pltpu.sync_copy(buf0, acc.at[idx0.at[0]], add=True)

            plsc.subcore_barrier()
            pltpu.sync_copy(acc.at[pl.ds(r0, _WB)],
                            out_hbm.at[pl.ds(c * N + r0, _WB)])

            @pl.when(sid == NSC - 1)
            def _():
                pltpu.sync_copy(acc.at[pl.ds(tail, ntail)],
                                out_hbm.at[pl.ds(c * N + tail, ntail)])

            plsc.subcore_barrier()

        scatter_phase(msg_hbm, omsg_hbm)
        scatter_phase(pm_hbm, opm_hbm)

        @pl.when(sid == 0)
        def _():
            pltpu.sync_copy(zm_hbm, acc)

        plsc.subcore_barrier()

        @pl.loop(0, NCHUNK // 2)
        def _(i):
            c0 = cbase + 2 * i
            c1 = c0 + 1
            i0 = pltpu.async_copy(send_hbm.at[pl.ds(c0, 1)], idx0, si)
            i1 = pltpu.async_copy(send_hbm.at[pl.ds(c1, 1)], idx1, si)
            i0.wait()
            a0 = pltpu.async_copy(ones_v, acc.at[idx0.at[0]], sa, add=True)
            i1.wait()
            a1 = pltpu.async_copy(ones_v, acc.at[idx1.at[0]], sa, add=True)
            a0.wait()
            a1.wait()

        clast = cbase + NCHUNK - 1
        pltpu.sync_copy(send_hbm.at[pl.ds(clast, 1)], idx0)
        pltpu.sync_copy(ones_v, acc.at[idx0.at[0]], add=True)

        plsc.subcore_barrier()
        pltpu.sync_copy(acc.at[pl.ds(r0, _WB)], ocnt_hbm.at[pl.ds(c * N + r0, _WB)])

        @pl.when(sid == NSC - 1)
        def _():
            pltpu.sync_copy(acc.at[pl.ds(tail, ntail)],
                            ocnt_hbm.at[pl.ds(c * N + tail, ntail)])

    return k(msg, pm, rec2, send2, zmsg, ones_c)


BN = 2000
BE = 2560


def _full(shape):
    return pl.BlockSpec(shape, lambda i: tuple(0 for _ in shape))


def _pre_call(s, vf, pos, wsi, wsj, wbv, wp, b1):
    grid = (N // BN,)
    return pl.pallas_call(
        _pre_body,
        grid=grid,
        in_specs=[
            pl.BlockSpec((BN, NSF), lambda i: (i, 0)),
            pl.BlockSpec((BN, F), lambda i: (i, 0)),
            pl.BlockSpec((BN, 1), lambda i: (i, 0)),
            _full((NSF, NSF)), _full((F, F)), _full((F, F)),
            _full((1, NSF)), _full((1, NSF)),
        ],
        out_specs=[
            pl.BlockSpec((BN, NSF), lambda i: (i, 0)),
            pl.BlockSpec((BN, NSF), lambda i: (i, 0)),
        ],
        out_shape=[
            jax.ShapeDtypeStruct((N, NSF), jnp.int32),
            jax.ShapeDtypeStruct((N, NSF), jnp.int32),
        ],
    )(s, vf, pos, wsi, wsj, wbv, wp, b1)


def _edge_call(ts_e, tr_e, weg, w2, pw1, wpr, bias4):
    grid = (E // BE,)
    return pl.pallas_call(
        _edge_body,
        grid=grid,
        in_specs=[
            pl.BlockSpec((BE, NSF), lambda i: (i, 0)),
            pl.BlockSpec((BE, NSF), lambda i: (i, 0)),
            _full((NSF, NSF)), _full((NSF, NSF)), _full((NSF, NSF)),
            _full((NSF, NSF)), _full((4, NSF)),
        ],
        out_specs=[
            pl.BlockSpec((BE, NSF), lambda i: (i, 0)),
            pl.BlockSpec((BE, F), lambda i: (i, 0)),
        ],
        out_shape=[
            jax.ShapeDtypeStruct((E, NSF), jnp.float32),
            jax.ShapeDtypeStruct((E, F), jnp.float32),
        ],
        compiler_params=pltpu.CompilerParams(
            dimension_semantics=("arbitrary",)),
    )(ts_e, tr_e, weg, w2, pw1, wpr, bias4)


def _node_call(s, vf, msgp, pmp, cntp, u1a, u1b, u2, wls, wrs, wo1, wo2,
               g, vec4, blrs):
    grid = (N // BN,)
    return pl.pallas_call(
        _node_body,
        grid=grid,
        in_specs=[
            pl.BlockSpec((BN, NSF), lambda i: (i, 0)),
            pl.BlockSpec((BN, F), lambda i: (i, 0)),
            pl.BlockSpec((NC, BN, NSF), lambda i: (0, i, 0)),
            pl.BlockSpec((NC, BN, F), lambda i: (0, i, 0)),
            pl.BlockSpec((NC, BN, NSF), lambda i: (0, i, 0)),
            _full((NSF, NSF)), _full((NSF, NSF)), _full((NSF, NSF)),
            _full((NSF, 8 * F)), _full((NSF, 8 * F)),
            _full((F, F)), _full((F, F)),
            _full((F, NV)), _full((4, NSF)), _full((2, 8 * F)),
        ],
        out_specs=[
            pl.BlockSpec((BN, NSF), lambda i: (i, 0)),
            pl.BlockSpec((BN, F), lambda i: (i, 0)),
        ],
        out_shape=[
            jax.ShapeDtypeStruct((N, NSF), jnp.float32),
            jax.ShapeDtypeStruct((N, F), jnp.float32),
        ],
    )(s, vf, msgp, pmp, cntp, u1a, u1b, u2, wls, wrs, wo1, wo2, g, vec4, blrs)


def kernel(s, v, positions, edge_index, v_w, v_b, msg_w1, msg_b1, msg_w2,
           msg_b2, pos_w1, pos_b1, pos_w2, pos_b2, upd_w1, upd_b1, upd_w2,
           upd_b2, gp_left_w, gp_left_b, gp_right_w, gp_right_b, gp_out_w,
           gp_out_b, gp_norm_a):
    f32 = jnp.float32
    send = edge_index[0]
    rec = edge_index[1]
    vf = v.reshape(N, F)
    pos = positions.reshape(N, 1)

    wsi = msg_w1[:, :NSF].T
    wsj = msg_w1[:, NSF:2 * NSF].T
    we_t = msg_w1[:, 2 * NSF:2 * NSF + NV].T
    wp = msg_w1[:, 2 * NSF + NV].reshape(1, NSF)
    weg = jnp.asarray(_G_SUM) @ we_t
    wbv = _mv_big(v_w)
    bv = _bias_flat(v_b)
    wpr = pos_w2.T @ jnp.asarray(_R_EXP)
    bpr = pos_b2 @ jnp.asarray(_R_EXP)
    bias4 = jnp.stack([bv, msg_b2, pos_b1, bpr], axis=0)

    wbl = _mv_big(gp_left_w)
    bl = _bias_flat(gp_left_b)
    wbr = _mv_big(gp_right_w)
    br = _bias_flat(gp_right_b)
    esel = jnp.asarray(_E_SEL)
    mmix = jnp.asarray(_M_MIX)
    wls = jnp.einsum('pq,iqr->pir', wbl, esel).reshape(F, 8 * F)
    bls = jnp.einsum('q,iqr->ir', bl, esel).reshape(8 * F)
    wrs = jnp.einsum('pq,iqr->pir', wbr, mmix).reshape(F, 8 * F)
    brs = jnp.einsum('q,iqr->ir', br, mmix).reshape(8 * F)
    blrs = jnp.stack([bls, brs], axis=0)

    wbo = _mv_big(gp_out_w)
    wo1 = wbo[:F]
    wo2 = wbo[F:]
    bo = _bias_flat(gp_out_b)
    arep = jnp.repeat(gp_norm_a, 8)
    vec4 = jnp.stack([upd_b1, upd_b2, bo, arep], axis=0)
    u1a = upd_w1[:, :NSF].T
    u1b = upd_w1[:, NSF:].T
    u2 = upd_w2.T

    ts, tr = _pre_call(s, vf, pos, wsi, wsj, wbv, wp,
                       msg_b1.reshape(1, NSF))

    rec2 = rec.reshape(E // ECH, ECH)
    send2 = send.reshape(E // ECH, ECH)
    ts_e, tr_e = _gather_call(ts, tr, send, rec)

    bf16 = jnp.bfloat16
    msg_e, pm_e = _edge_call(ts_e, tr_e, weg.astype(bf16),
                             msg_w2.T.astype(bf16), pos_w1.T.astype(bf16),
                             wpr.astype(bf16), bias4)

    zmsg = jnp.zeros((N, NSF), f32)
    ones_c = jnp.ones((ECH, NSF), f32)
    msg_part, pm_part, cnt_part = _scatter_call(
        msg_e, pm_e, rec2, send2, zmsg, ones_c)

    s_out, v_out = _node_call(
        s, vf,
        msg_part.reshape(NC, N, NSF),
        pm_part.reshape(NC, N, F),
        cnt_part.reshape(NC, N, NSF),
        u1a, u1b, u2, wls, wrs, wo1, wo2,
        jnp.asarray(_G_SUM), vec4, blrs)

    return (s_out, v_out.reshape(N, NV, 8))
